# Initial kernel scaffold; baseline (speedup 1.0000x reference)
import jax
import jax.numpy as jnp
from jax.experimental import pallas as pl


def kernel(x, edge_index, Wp, bp, convW, convb, bn_w, bn_b, bn_mean, bn_var, ln_w, ln_b, mlpW1, mlpb1, mlpW2, mlpb2, outW, outb):
    raise NotImplementedError("write your pallas kernel here")


# speedup vs baseline: 1.0375x; 1.0375x over previous
import functools

import jax
import jax.numpy as jnp
from jax.experimental import pallas as pl
from jax.experimental.pallas import tpu as pltpu

N = 10000
D_IN = 128
D_H = 256
L = 4

_BLK = 1000


def _proj_body(x_ref, w_ref, b_ref, o_ref):
    o_ref[...] = jnp.maximum(
        jnp.dot(x_ref[...], w_ref[...], preferred_element_type=jnp.float32)
        + b_ref[...], 0.0)


def _proj(x, W, b):
    m, k = x.shape
    n = W.shape[1]
    return pl.pallas_call(
        _proj_body,
        grid=(m // _BLK,),
        in_specs=[
            pl.BlockSpec((_BLK, k), lambda i: (i, 0)),
            pl.BlockSpec((k, n), lambda i: (0, 0)),
            pl.BlockSpec((1, n), lambda i: (0, 0)),
        ],
        out_specs=pl.BlockSpec((_BLK, n), lambda i: (i, 0)),
        out_shape=jax.ShapeDtypeStruct((m, n), jnp.float32),
    )(x, W, b.reshape(1, -1))


def _head_body(h_ref, w1_ref, b1_ref, w2_ref, b2_ref, wo_ref, bo_ref, o_ref):
    h = h_ref[...]
    t = jnp.maximum(
        jnp.dot(h, w1_ref[...], preferred_element_type=jnp.float32)
        + b1_ref[...], 0.0)
    t = jnp.dot(t, w2_ref[...], preferred_element_type=jnp.float32) + b2_ref[...]
    o = jnp.dot(t, wo_ref[...], preferred_element_type=jnp.float32) + bo_ref[...]
    m = jnp.max(o, axis=1, keepdims=True)
    s = jnp.log(jnp.sum(jnp.exp(o - m), axis=1, keepdims=True))
    o_ref[...] = o - m - s


def _head(h, W1, b1, W2, b2, Wo, bo):
    m = h.shape[0]
    d_out = Wo.shape[1]
    return pl.pallas_call(
        _head_body,
        grid=(m // _BLK,),
        in_specs=[
            pl.BlockSpec((_BLK, D_H), lambda i: (i, 0)),
            pl.BlockSpec((D_H, 2 * D_H), lambda i: (0, 0)),
            pl.BlockSpec((1, 2 * D_H), lambda i: (0, 0)),
            pl.BlockSpec((2 * D_H, D_H), lambda i: (0, 0)),
            pl.BlockSpec((1, D_H), lambda i: (0, 0)),
            pl.BlockSpec((D_H, d_out), lambda i: (0, 0)),
            pl.BlockSpec((1, d_out), lambda i: (0, 0)),
        ],
        out_specs=pl.BlockSpec((_BLK, d_out), lambda i: (i, 0)),
        out_shape=jax.ShapeDtypeStruct((m, d_out), jnp.float32),
    )(h, W1, b1.reshape(1, -1), W2, b2.reshape(1, -1), Wo, bo.reshape(1, -1))


def kernel(x, edge_index, Wp, bp, convW, convb, bn_w, bn_b, bn_mean, bn_var,
           ln_w, ln_b, mlpW1, mlpb1, mlpW2, mlpb2, outW, outb):
    n = x.shape[0]
    loop = jnp.arange(n, dtype=edge_index.dtype)
    src = jnp.concatenate([edge_index[0], loop])
    dst = jnp.concatenate([edge_index[1], loop])
    deg = jnp.zeros((n,), dtype=x.dtype).at[dst].add(1.0)
    dinv = jnp.where(deg > 0, deg ** -0.5, 0.0)
    norm = dinv[src] * dinv[dst]

    h = _proj(x, Wp, bp)
    for i in range(L):
        h_prev = h
        hw = h @ convW[i]
        msg = hw[src] * norm[:, None]
        h = jnp.zeros_like(hw).at[dst].add(msg) + convb[i]
        h = (h - bn_mean[i]) / jnp.sqrt(bn_var[i] + 1e-5) * bn_w[i] + bn_b[i]
        mu = h.mean(axis=1, keepdims=True)
        var = h.var(axis=1, keepdims=True)
        h = (h - mu) / jnp.sqrt(var + 1e-5) * ln_w[i] + ln_b[i]
        h = jnp.maximum(h, 0.0)
        h = h + h_prev
    return _head(h, mlpW1, mlpb1, mlpW2, mlpb2, outW, outb)


# speedup vs baseline: 10.7264x; 10.3384x over previous
import functools

import jax
import jax.numpy as jnp
from jax import lax
from jax.experimental import pallas as pl
from jax.experimental.pallas import tpu as pltpu
from jax.experimental.pallas import tpu_sc as plsc

N = 10000
E = 320000
D_IN = 128
D_H = 256
HALF = 128
L = 4

NC, NS = 2, 16
CHUNK = 128
E_PAD = 327680
NCHUNK = E_PAD // CHUNK
CPT = NCHUNK // NS
CPW = NCHUNK // (NC * NS)
ACC_ROWS = N + 112
ZROWS = ACC_ROWS // NS
SUP = 8

_mesh = plsc.VectorSubcoreMesh(core_axis_name="c", subcore_axis_name="s")


def _deg_body(dstc_hbm, ones_hbm, zeros_hbm, out_hbm, acc, dst_v, ones_v):
    c = lax.axis_index("c")
    s = lax.axis_index("s")
    pltpu.sync_copy(zeros_hbm.at[pl.ds(s * ZROWS, ZROWS)],
                    acc.at[pl.ds(s * ZROWS, ZROWS)])
    pltpu.sync_copy(ones_hbm, ones_v)
    w = c * NS + s
    base = w * CPW
    plsc.subcore_barrier()

    def body(g, carry):
        pltpu.sync_copy(dstc_hbm.at[pl.ds(base + g * SUP, SUP)], dst_v)
        for j in range(SUP):
            pltpu.sync_copy(ones_v, acc.at[dst_v.at[j]], add=True)
        return carry

    lax.fori_loop(0, CPW // SUP, body, 0)
    plsc.subcore_barrier()
    pltpu.sync_copy(acc.at[pl.ds(s * ZROWS, ZROWS)],
                    out_hbm.at[c, pl.ds(s * ZROWS, ZROWS)])


_deg = pl.kernel(
    _deg_body,
    out_type=jax.ShapeDtypeStruct((NC, ACC_ROWS, HALF), jnp.float32),
    name="gcn_deg_sc",
    mesh=_mesh,
    scratch_types=[
        pltpu.VMEM_SHARED((ACC_ROWS, HALF), jnp.float32),
        pltpu.VMEM((SUP, CHUNK), jnp.int32),
        pltpu.VMEM((CHUNK, HALF), jnp.float32),
    ],
)


def _agg_body(hwp_hbm, srcc_hbm, dstc_hbm, zeros_hbm, out_hbm,
              acc, src_v, dst_v, stage, sem):
    c = lax.axis_index("c")
    s = lax.axis_index("s")
    pltpu.sync_copy(zeros_hbm.at[pl.ds(s * ZROWS, ZROWS)],
                    acc.at[pl.ds(s * ZROWS, ZROWS)])
    base = s * CPT
    plsc.subcore_barrier()

    def body(g, carry):
        pltpu.sync_copy(srcc_hbm.at[c, pl.ds(base + g * SUP, SUP)], src_v)
        pltpu.sync_copy(dstc_hbm.at[pl.ds(base + g * SUP, SUP)], dst_v)
        for j in range(SUP):
            pltpu.async_copy(hwp_hbm.at[src_v.at[j]], stage, sem).wait()
            pltpu.sync_copy(stage, acc.at[dst_v.at[j]], add=True)
        return carry

    lax.fori_loop(0, CPT // SUP, body, 0)
    plsc.subcore_barrier()
    pltpu.sync_copy(acc.at[pl.ds(s * ZROWS, ZROWS)],
                    out_hbm.at[c, pl.ds(s * ZROWS, ZROWS)])


_agg = pl.kernel(
    _agg_body,
    out_type=jax.ShapeDtypeStruct((NC, ACC_ROWS, HALF), jnp.float32),
    name="gcn_agg_sc",
    mesh=_mesh,
    scratch_types=[
        pltpu.VMEM_SHARED((ACC_ROWS, HALF), jnp.float32),
        pltpu.VMEM((SUP, CHUNK), jnp.int32),
        pltpu.VMEM((SUP, CHUNK), jnp.int32),
        pltpu.VMEM((CHUNK, HALF), jnp.float32),
        pltpu.SemaphoreType.DMA,
    ],
)


_BLK = 1000


def _proj_body(x_ref, w_ref, b_ref, o_ref):
    o_ref[...] = jnp.maximum(
        jnp.dot(x_ref[...], w_ref[...], preferred_element_type=jnp.float32)
        + b_ref[...], 0.0)


def _proj(x, W, b):
    m, k = x.shape
    n = W.shape[1]
    return pl.pallas_call(
        _proj_body,
        grid=(m // _BLK,),
        in_specs=[
            pl.BlockSpec((_BLK, k), lambda i: (i, 0)),
            pl.BlockSpec((k, n), lambda i: (0, 0)),
            pl.BlockSpec((1, n), lambda i: (0, 0)),
        ],
        out_specs=pl.BlockSpec((_BLK, n), lambda i: (i, 0)),
        out_shape=jax.ShapeDtypeStruct((m, n), jnp.float32),
    )(x, W, b.reshape(1, -1))


def _head_body(h_ref, w1_ref, b1_ref, w2_ref, b2_ref, wo_ref, bo_ref, o_ref):
    h = h_ref[...]
    t = jnp.maximum(
        jnp.dot(h, w1_ref[...], preferred_element_type=jnp.float32)
        + b1_ref[...], 0.0)
    t = jnp.dot(t, w2_ref[...], preferred_element_type=jnp.float32) + b2_ref[...]
    o = jnp.dot(t, wo_ref[...], preferred_element_type=jnp.float32) + bo_ref[...]
    m = jnp.max(o, axis=1, keepdims=True)
    lse = jnp.log(jnp.sum(jnp.exp(o - m), axis=1, keepdims=True))
    o_ref[...] = o - m - lse


def _head(h, W1, b1, W2, b2, Wo, bo):
    m = h.shape[0]
    d_out = Wo.shape[1]
    return pl.pallas_call(
        _head_body,
        grid=(m // _BLK,),
        in_specs=[
            pl.BlockSpec((_BLK, D_H), lambda i: (i, 0)),
            pl.BlockSpec((D_H, 2 * D_H), lambda i: (0, 0)),
            pl.BlockSpec((1, 2 * D_H), lambda i: (0, 0)),
            pl.BlockSpec((2 * D_H, D_H), lambda i: (0, 0)),
            pl.BlockSpec((1, D_H), lambda i: (0, 0)),
            pl.BlockSpec((D_H, d_out), lambda i: (0, 0)),
            pl.BlockSpec((1, d_out), lambda i: (0, 0)),
        ],
        out_specs=pl.BlockSpec((_BLK, d_out), lambda i: (i, 0)),
        out_shape=jax.ShapeDtypeStruct((m, d_out), jnp.float32),
    )(h, W1, b1.reshape(1, -1), W2, b2.reshape(1, -1), Wo, bo.reshape(1, -1))


def kernel(x, edge_index, Wp, bp, convW, convb, bn_w, bn_b, bn_mean, bn_var,
           ln_w, ln_b, mlpW1, mlpb1, mlpW2, mlpb2, outW, outb):
    src = edge_index[0]
    dst = edge_index[1]
    pad = E_PAD - E
    src_pad = jnp.concatenate(
        [src, (jnp.arange(pad, dtype=jnp.int32) * 97) % N])
    dst_pad = jnp.concatenate(
        [dst, N + (jnp.arange(pad, dtype=jnp.int32) % 16)])
    dst_chunks = dst_pad.reshape(NCHUNK, CHUNK)
    src_chunks2 = jnp.stack([src_pad, src_pad + N]).reshape(NC, NCHUNK, CHUNK)
    ones128 = jnp.ones((CHUNK, HALF), jnp.float32)
    zeros128 = jnp.zeros((ACC_ROWS, HALF), jnp.float32)

    deg_pl = _deg(dst_chunks, ones128, zeros128)
    deg = deg_pl[0, :N, 0] + deg_pl[1, :N, 0] + 1.0
    dinv = deg ** -0.5

    h = _proj(x, Wp, bp)
    for i in range(L):
        h_prev = h
        hw = h @ convW[i]
        hwp = hw * dinv[:, None]
        hwp_flat = jnp.concatenate([hwp[:, :HALF], hwp[:, HALF:]], axis=0)
        agg_pl = _agg(hwp_flat, src_chunks2, dst_chunks, zeros128)
        agg = jnp.concatenate([agg_pl[0, :N], agg_pl[1, :N]], axis=1)
        h = dinv[:, None] * (agg + hwp) + convb[i]
        h = (h - bn_mean[i]) / jnp.sqrt(bn_var[i] + 1e-5) * bn_w[i] + bn_b[i]
        mu = h.mean(axis=1, keepdims=True)
        var = h.var(axis=1, keepdims=True)
        h = (h - mu) / jnp.sqrt(var + 1e-5) * ln_w[i] + ln_b[i]
        h = jnp.maximum(h, 0.0)
        h = h + h_prev
    return _head(h, mlpW1, mlpb1, mlpW2, mlpb2, outW, outb)


# speedup vs baseline: 15.5361x; 1.4484x over previous
import functools

import jax
import jax.numpy as jnp
from jax import lax
from jax.experimental import pallas as pl
from jax.experimental.pallas import tpu as pltpu
from jax.experimental.pallas import tpu_sc as plsc

N = 10000
E = 320000
D_IN = 128
D_H = 256
HALF = 128
L = 4

NC, NS = 2, 16
CHUNK = 128
E_PAD = 327680
NCHUNK = E_PAD // CHUNK
CPT = NCHUNK // NS
CPW = NCHUNK // (NC * NS)
ACC_ROWS = N + 112
ZROWS = ACC_ROWS // NS
SUP = 16

_mesh = plsc.VectorSubcoreMesh(core_axis_name="c", subcore_axis_name="s")


def _deg_body(dstc_hbm, ones_hbm, zeros_hbm, out_hbm, acc, dst_v, ones_v):
    c = lax.axis_index("c")
    s = lax.axis_index("s")
    pltpu.sync_copy(zeros_hbm.at[pl.ds(s * ZROWS, ZROWS)],
                    acc.at[pl.ds(s * ZROWS, ZROWS)])
    pltpu.sync_copy(ones_hbm, ones_v)
    w = c * NS + s
    base = w * CPW
    plsc.subcore_barrier()

    def body(g, carry):
        pltpu.sync_copy(dstc_hbm.at[pl.ds(base + g * SUP, SUP)], dst_v)
        for j in range(SUP):
            pltpu.sync_copy(ones_v, acc.at[dst_v.at[j]], add=True)
        return carry

    lax.fori_loop(0, CPW // SUP, body, 0)
    plsc.subcore_barrier()
    pltpu.sync_copy(acc.at[pl.ds(s * ZROWS, ZROWS)],
                    out_hbm.at[c, pl.ds(s * ZROWS, ZROWS)])


_deg = pl.kernel(
    _deg_body,
    out_type=jax.ShapeDtypeStruct((NC, ACC_ROWS, HALF), jnp.float32),
    name="gcn_deg_sc",
    mesh=_mesh,
    scratch_types=[
        pltpu.VMEM_SHARED((ACC_ROWS, HALF), jnp.float32),
        pltpu.VMEM((SUP, CHUNK), jnp.int32),
        pltpu.VMEM((CHUNK, HALF), jnp.float32),
    ],
)


def _agg_body(hwp_hbm, srcc_hbm, dstc_hbm, zeros_hbm, out_hbm,
              acc, src_v, dst_v, stage0, stage1, sem0, sem1):
    c = lax.axis_index("c")
    s = lax.axis_index("s")
    pltpu.sync_copy(zeros_hbm.at[pl.ds(s * ZROWS, ZROWS)],
                    acc.at[pl.ds(s * ZROWS, ZROWS)])
    base = s * CPT
    plsc.subcore_barrier()
    stages = (stage0, stage1)
    sems = (sem0, sem1)

    def body(g, carry):
        pltpu.sync_copy(srcc_hbm.at[c, pl.ds(base + g * SUP, SUP)], src_v)
        pltpu.sync_copy(dstc_hbm.at[pl.ds(base + g * SUP, SUP)], dst_v)
        pend = pltpu.async_copy(hwp_hbm.at[src_v.at[0]], stages[0], sems[0])
        for j in range(SUP):
            if j + 1 < SUP:
                nxt = pltpu.async_copy(hwp_hbm.at[src_v.at[j + 1]],
                                       stages[(j + 1) % 2], sems[(j + 1) % 2])
            pend.wait()
            pltpu.sync_copy(stages[j % 2], acc.at[dst_v.at[j]], add=True)
            if j + 1 < SUP:
                pend = nxt
        return carry

    lax.fori_loop(0, CPT // SUP, body, 0)
    plsc.subcore_barrier()
    pltpu.sync_copy(acc.at[pl.ds(s * ZROWS, ZROWS)],
                    out_hbm.at[c, pl.ds(s * ZROWS, ZROWS)])


_agg = pl.kernel(
    _agg_body,
    out_type=jax.ShapeDtypeStruct((NC, ACC_ROWS, HALF), jnp.float32),
    name="gcn_agg_sc",
    mesh=_mesh,
    scratch_types=[
        pltpu.VMEM_SHARED((ACC_ROWS, HALF), jnp.float32),
        pltpu.VMEM((SUP, CHUNK), jnp.int32),
        pltpu.VMEM((SUP, CHUNK), jnp.int32),
        pltpu.VMEM((CHUNK, HALF), jnp.float32),
        pltpu.VMEM((CHUNK, HALF), jnp.float32),
        pltpu.SemaphoreType.DMA,
        pltpu.SemaphoreType.DMA,
    ],
)


_BLK = 1000


def _proj_body(x_ref, w_ref, b_ref, o_ref):
    o_ref[...] = jnp.maximum(
        jnp.dot(x_ref[...], w_ref[...], preferred_element_type=jnp.float32)
        + b_ref[...], 0.0)


def _proj(x, W, b):
    m, k = x.shape
    n = W.shape[1]
    return pl.pallas_call(
        _proj_body,
        grid=(m // _BLK,),
        in_specs=[
            pl.BlockSpec((_BLK, k), lambda i: (i, 0)),
            pl.BlockSpec((k, n), lambda i: (0, 0)),
            pl.BlockSpec((1, n), lambda i: (0, 0)),
        ],
        out_specs=pl.BlockSpec((_BLK, n), lambda i: (i, 0)),
        out_shape=jax.ShapeDtypeStruct((m, n), jnp.float32),
    )(x, W, b.reshape(1, -1))


def _head_body(h_ref, w1_ref, b1_ref, w2_ref, b2_ref, wo_ref, bo_ref, o_ref):
    h = h_ref[...]
    t = jnp.maximum(
        jnp.dot(h, w1_ref[...], preferred_element_type=jnp.float32)
        + b1_ref[...], 0.0)
    t = jnp.dot(t, w2_ref[...], preferred_element_type=jnp.float32) + b2_ref[...]
    o = jnp.dot(t, wo_ref[...], preferred_element_type=jnp.float32) + bo_ref[...]
    m = jnp.max(o, axis=1, keepdims=True)
    lse = jnp.log(jnp.sum(jnp.exp(o - m), axis=1, keepdims=True))
    o_ref[...] = o - m - lse


def _head(h, W1, b1, W2, b2, Wo, bo):
    m = h.shape[0]
    d_out = Wo.shape[1]
    return pl.pallas_call(
        _head_body,
        grid=(m // _BLK,),
        in_specs=[
            pl.BlockSpec((_BLK, D_H), lambda i: (i, 0)),
            pl.BlockSpec((D_H, 2 * D_H), lambda i: (0, 0)),
            pl.BlockSpec((1, 2 * D_H), lambda i: (0, 0)),
            pl.BlockSpec((2 * D_H, D_H), lambda i: (0, 0)),
            pl.BlockSpec((1, D_H), lambda i: (0, 0)),
            pl.BlockSpec((D_H, d_out), lambda i: (0, 0)),
            pl.BlockSpec((1, d_out), lambda i: (0, 0)),
        ],
        out_specs=pl.BlockSpec((_BLK, d_out), lambda i: (i, 0)),
        out_shape=jax.ShapeDtypeStruct((m, d_out), jnp.float32),
    )(h, W1, b1.reshape(1, -1), W2, b2.reshape(1, -1), Wo, bo.reshape(1, -1))


def kernel(x, edge_index, Wp, bp, convW, convb, bn_w, bn_b, bn_mean, bn_var,
           ln_w, ln_b, mlpW1, mlpb1, mlpW2, mlpb2, outW, outb):
    src = edge_index[0]
    dst = edge_index[1]
    pad = E_PAD - E
    src_pad = jnp.concatenate(
        [src, (jnp.arange(pad, dtype=jnp.int32) * 97) % N])
    dst_pad = jnp.concatenate(
        [dst, N + (jnp.arange(pad, dtype=jnp.int32) % 16)])
    dst_chunks = dst_pad.reshape(NCHUNK, CHUNK)
    src_chunks2 = jnp.stack([src_pad, src_pad + N]).reshape(NC, NCHUNK, CHUNK)
    ones128 = jnp.ones((CHUNK, HALF), jnp.float32)
    zeros128 = jnp.zeros((ACC_ROWS, HALF), jnp.float32)

    deg_pl = _deg(dst_chunks, ones128, zeros128)
    deg = deg_pl[0, :N, 0] + deg_pl[1, :N, 0] + 1.0
    dinv = deg ** -0.5

    h = _proj(x, Wp, bp)
    for i in range(L):
        h_prev = h
        hw = h @ convW[i]
        hwp = hw * dinv[:, None]
        hwp_flat = jnp.concatenate([hwp[:, :HALF], hwp[:, HALF:]], axis=0)
        agg_pl = _agg(hwp_flat, src_chunks2, dst_chunks, zeros128)
        agg = jnp.concatenate([agg_pl[0, :N], agg_pl[1, :N]], axis=1)
        h = dinv[:, None] * (agg + hwp) + convb[i]
        h = (h - bn_mean[i]) / jnp.sqrt(bn_var[i] + 1e-5) * bn_w[i] + bn_b[i]
        mu = h.mean(axis=1, keepdims=True)
        var = h.var(axis=1, keepdims=True)
        h = (h - mu) / jnp.sqrt(var + 1e-5) * ln_w[i] + ln_b[i]
        h = jnp.maximum(h, 0.0)
        h = h + h_prev
    return _head(h, mlpW1, mlpb1, mlpW2, mlpb2, outW, outb)


# speedup vs baseline: 16.9138x; 1.0887x over previous
import functools

import jax
import jax.numpy as jnp
from jax import lax
from jax.experimental import pallas as pl
from jax.experimental.pallas import tpu as pltpu
from jax.experimental.pallas import tpu_sc as plsc

N = 10000
E = 320000
D_IN = 128
D_H = 256
HALF = 128
L = 4

NC, NS = 2, 16
CHUNK = 128
E_PAD = 327680
NCHUNK = E_PAD // CHUNK
CPT = NCHUNK // NS
CPW = NCHUNK // (NC * NS)
ACC_ROWS = N + 112
ZROWS = ACC_ROWS // NS
SUP = 16

_mesh = plsc.VectorSubcoreMesh(core_axis_name="c", subcore_axis_name="s")


def _deg_body(dstc_hbm, ones_hbm, zeros_hbm, out_hbm, acc, dst_v, ones_v):
    c = lax.axis_index("c")
    s = lax.axis_index("s")
    pltpu.sync_copy(zeros_hbm.at[pl.ds(s * ZROWS, ZROWS)],
                    acc.at[pl.ds(s * ZROWS, ZROWS)])
    pltpu.sync_copy(ones_hbm, ones_v)
    w = c * NS + s
    base = w * CPW
    plsc.subcore_barrier()

    def body(g, carry):
        pltpu.sync_copy(dstc_hbm.at[pl.ds(base + g * SUP, SUP)], dst_v)
        for j in range(SUP):
            pltpu.sync_copy(ones_v, acc.at[dst_v.at[j]], add=True)
        return carry

    lax.fori_loop(0, CPW // SUP, body, 0)
    plsc.subcore_barrier()
    pltpu.sync_copy(acc.at[pl.ds(s * ZROWS, ZROWS)],
                    out_hbm.at[c, pl.ds(s * ZROWS, ZROWS)])


_deg = pl.kernel(
    _deg_body,
    out_type=jax.ShapeDtypeStruct((NC, ACC_ROWS, HALF), jnp.float32),
    name="gcn_deg_sc",
    mesh=_mesh,
    scratch_types=[
        pltpu.VMEM_SHARED((ACC_ROWS, HALF), jnp.float32),
        pltpu.VMEM((SUP, CHUNK), jnp.int32),
        pltpu.VMEM((CHUNK, HALF), jnp.float32),
    ],
)


def _agg_body(hwp_hbm, srcc_hbm, dstc_hbm, zeros_hbm, out_hbm,
              acc, src_v, dst_v, stage0, stage1, sem0, sem1):
    c = lax.axis_index("c")
    s = lax.axis_index("s")
    pltpu.sync_copy(zeros_hbm.at[pl.ds(s * ZROWS, ZROWS)],
                    acc.at[pl.ds(s * ZROWS, ZROWS)])
    base = s * CPT
    plsc.subcore_barrier()
    stages = (stage0, stage1)
    sems = (sem0, sem1)

    def body(g, carry):
        pltpu.sync_copy(srcc_hbm.at[c, pl.ds(base + g * SUP, SUP)], src_v)
        pltpu.sync_copy(dstc_hbm.at[pl.ds(base + g * SUP, SUP)], dst_v)
        pend = pltpu.async_copy(hwp_hbm.at[src_v.at[0]], stages[0], sems[0])
        for j in range(SUP):
            if j + 1 < SUP:
                nxt = pltpu.async_copy(hwp_hbm.at[src_v.at[j + 1]],
                                       stages[(j + 1) % 2], sems[(j + 1) % 2])
            pend.wait()
            pltpu.sync_copy(stages[j % 2], acc.at[dst_v.at[j]], add=True)
            if j + 1 < SUP:
                pend = nxt
        return carry

    lax.fori_loop(0, CPT // SUP, body, 0)
    plsc.subcore_barrier()
    pltpu.sync_copy(acc.at[pl.ds(s * ZROWS, ZROWS)],
                    out_hbm.at[c, pl.ds(s * ZROWS, ZROWS)])


_agg = pl.kernel(
    _agg_body,
    out_type=jax.ShapeDtypeStruct((NC, ACC_ROWS, HALF), jnp.float32),
    name="gcn_agg_sc",
    mesh=_mesh,
    scratch_types=[
        pltpu.VMEM_SHARED((ACC_ROWS, HALF), jnp.float32),
        pltpu.VMEM((SUP, CHUNK), jnp.int32),
        pltpu.VMEM((SUP, CHUNK), jnp.int32),
        pltpu.VMEM((CHUNK, HALF), jnp.float32),
        pltpu.VMEM((CHUNK, HALF), jnp.float32),
        pltpu.SemaphoreType.DMA,
        pltpu.SemaphoreType.DMA,
    ],
)


_BLK = 1000


def _split_store(o_ref, hw):
    o_ref[0, :, :] = hw[:, :HALF]
    o_ref[1, :, :] = hw[:, HALF:]


def _proj_pre_body(x_ref, wp_ref, bp_ref, w0_ref, dsum_ref, h_ref, hwp_ref):
    h = jnp.maximum(
        jnp.dot(x_ref[...], wp_ref[...], preferred_element_type=jnp.float32)
        + bp_ref[...], 0.0)
    h_ref[...] = h
    dinv = lax.rsqrt(dsum_ref[...])
    hw = jnp.dot(h, w0_ref[...], preferred_element_type=jnp.float32) * dinv
    _split_store(hwp_ref, hw)


def _proj_pre(x, Wp, bp, W0, dsum):
    return pl.pallas_call(
        _proj_pre_body,
        grid=(N // _BLK,),
        in_specs=[
            pl.BlockSpec((_BLK, D_IN), lambda i: (i, 0)),
            pl.BlockSpec((D_IN, D_H), lambda i: (0, 0)),
            pl.BlockSpec((1, D_H), lambda i: (0, 0)),
            pl.BlockSpec((D_H, D_H), lambda i: (0, 0)),
            pl.BlockSpec((_BLK, 1), lambda i: (i, 0)),
        ],
        out_specs=(
            pl.BlockSpec((_BLK, D_H), lambda i: (i, 0)),
            pl.BlockSpec((2, _BLK, HALF), lambda i: (0, i, 0)),
        ),
        out_shape=(
            jax.ShapeDtypeStruct((N, D_H), jnp.float32),
            jax.ShapeDtypeStruct((2, N, HALF), jnp.float32),
        ),
    )(x, Wp, bp.reshape(1, -1), W0, dsum)


def _epilogue(agg_ref, hwp_ref, dsum_ref, bnsc_ref, bnsh_ref, lnw_ref,
              lnb_ref, hprev_ref):
    dinv = lax.rsqrt(dsum_ref[...])
    agg = jnp.concatenate([agg_ref[0], agg_ref[1]], axis=1)
    hwp = jnp.concatenate([hwp_ref[0], hwp_ref[1]], axis=1)
    t = dinv * (agg + hwp) * bnsc_ref[...] + bnsh_ref[...]
    mu = jnp.mean(t, axis=1, keepdims=True)
    var = jnp.mean(t * t, axis=1, keepdims=True) - mu * mu
    t = (t - mu) * lax.rsqrt(var + 1e-5) * lnw_ref[...] + lnb_ref[...]
    return jnp.maximum(t, 0.0) + hprev_ref[...], dinv


def _post_pre_body(agg_ref, hwp_ref, dsum_ref, bnsc_ref, bnsh_ref, lnw_ref,
                   lnb_ref, hprev_ref, wn_ref, h_ref, hwpn_ref):
    h, dinv = _epilogue(agg_ref, hwp_ref, dsum_ref, bnsc_ref, bnsh_ref,
                        lnw_ref, lnb_ref, hprev_ref)
    h_ref[...] = h
    hw = jnp.dot(h, wn_ref[...], preferred_element_type=jnp.float32) * dinv
    _split_store(hwpn_ref, hw)


_EPI_SPECS = [
    pl.BlockSpec((2, _BLK, HALF), lambda i: (0, i, 0)),
    pl.BlockSpec((2, _BLK, HALF), lambda i: (0, i, 0)),
    pl.BlockSpec((_BLK, 1), lambda i: (i, 0)),
    pl.BlockSpec((1, D_H), lambda i: (0, 0)),
    pl.BlockSpec((1, D_H), lambda i: (0, 0)),
    pl.BlockSpec((1, D_H), lambda i: (0, 0)),
    pl.BlockSpec((1, D_H), lambda i: (0, 0)),
    pl.BlockSpec((_BLK, D_H), lambda i: (i, 0)),
]


def _post_pre(agg_pl, hwp_pl, dsum, bnsc, bnsh, lnw, lnb, hprev, Wn):
    return pl.pallas_call(
        _post_pre_body,
        grid=(N // _BLK,),
        in_specs=_EPI_SPECS + [pl.BlockSpec((D_H, D_H), lambda i: (0, 0))],
        out_specs=(
            pl.BlockSpec((_BLK, D_H), lambda i: (i, 0)),
            pl.BlockSpec((2, _BLK, HALF), lambda i: (0, i, 0)),
        ),
        out_shape=(
            jax.ShapeDtypeStruct((N, D_H), jnp.float32),
            jax.ShapeDtypeStruct((2, N, HALF), jnp.float32),
        ),
    )(agg_pl, hwp_pl, dsum, bnsc.reshape(1, -1), bnsh.reshape(1, -1),
      lnw.reshape(1, -1), lnb.reshape(1, -1), hprev, Wn)


def _post_head_body(agg_ref, hwp_ref, dsum_ref, bnsc_ref, bnsh_ref, lnw_ref,
                    lnb_ref, hprev_ref, w1_ref, b1_ref, w2_ref, b2_ref,
                    wo_ref, bo_ref, o_ref):
    h, _ = _epilogue(agg_ref, hwp_ref, dsum_ref, bnsc_ref, bnsh_ref,
                     lnw_ref, lnb_ref, hprev_ref)
    t = jnp.maximum(
        jnp.dot(h, w1_ref[...], preferred_element_type=jnp.float32)
        + b1_ref[...], 0.0)
    t = jnp.dot(t, w2_ref[...], preferred_element_type=jnp.float32) + b2_ref[...]
    o = jnp.dot(t, wo_ref[...], preferred_element_type=jnp.float32) + bo_ref[...]
    m = jnp.max(o, axis=1, keepdims=True)
    lse = jnp.log(jnp.sum(jnp.exp(o - m), axis=1, keepdims=True))
    o_ref[...] = o - m - lse


def _post_head(agg_pl, hwp_pl, dsum, bnsc, bnsh, lnw, lnb, hprev,
               W1, b1, W2, b2, Wo, bo):
    d_out = Wo.shape[1]
    return pl.pallas_call(
        _post_head_body,
        grid=(N // _BLK,),
        in_specs=_EPI_SPECS + [
            pl.BlockSpec((D_H, 2 * D_H), lambda i: (0, 0)),
            pl.BlockSpec((1, 2 * D_H), lambda i: (0, 0)),
            pl.BlockSpec((2 * D_H, D_H), lambda i: (0, 0)),
            pl.BlockSpec((1, D_H), lambda i: (0, 0)),
            pl.BlockSpec((D_H, d_out), lambda i: (0, 0)),
            pl.BlockSpec((1, d_out), lambda i: (0, 0)),
        ],
        out_specs=pl.BlockSpec((_BLK, d_out), lambda i: (i, 0)),
        out_shape=jax.ShapeDtypeStruct((N, d_out), jnp.float32),
    )(agg_pl, hwp_pl, dsum, bnsc.reshape(1, -1), bnsh.reshape(1, -1),
      lnw.reshape(1, -1), lnb.reshape(1, -1), hprev,
      W1, b1.reshape(1, -1), W2, b2.reshape(1, -1), Wo, bo.reshape(1, -1))


def kernel(x, edge_index, Wp, bp, convW, convb, bn_w, bn_b, bn_mean, bn_var,
           ln_w, ln_b, mlpW1, mlpb1, mlpW2, mlpb2, outW, outb):
    src = edge_index[0]
    dst = edge_index[1]
    pad = E_PAD - E
    src_pad = jnp.concatenate(
        [src, (jnp.arange(pad, dtype=jnp.int32) * 97) % N])
    dst_pad = jnp.concatenate(
        [dst, N + (jnp.arange(pad, dtype=jnp.int32) % 16)])
    dst_chunks = dst_pad.reshape(NCHUNK, CHUNK)
    src_chunks2 = jnp.stack([src_pad, src_pad + N]).reshape(NC, NCHUNK, CHUNK)
    ones128 = jnp.ones((CHUNK, HALF), jnp.float32)
    zeros128 = jnp.zeros((ACC_ROWS, HALF), jnp.float32)

    deg_pl = _deg(dst_chunks, ones128, zeros128)
    dsum = deg_pl[0, :N, 0:1] + deg_pl[1, :N, 0:1] + 1.0

    bn_sc = bn_w / jnp.sqrt(bn_var + 1e-5)
    bn_sh = (convb - bn_mean) * bn_sc + bn_b

    h, hwp_pl = _proj_pre(x, Wp, bp, convW[0], dsum)
    for i in range(L):
        agg_pl = _agg(hwp_pl.reshape(2 * N, HALF), src_chunks2, dst_chunks,
                      zeros128)
        if i < L - 1:
            h, hwp_pl = _post_pre(agg_pl, hwp_pl, dsum, bn_sc[i], bn_sh[i],
                                  ln_w[i], ln_b[i], h, convW[i + 1])
        else:
            out = _post_head(agg_pl, hwp_pl, dsum, bn_sc[i], bn_sh[i],
                             ln_w[i], ln_b[i], h, mlpW1, mlpb1, mlpW2, mlpb2,
                             outW, outb)
    return out


# speedup vs baseline: 16.9251x; 1.0007x over previous
import functools

import jax
import jax.numpy as jnp
from jax import lax
from jax.experimental import pallas as pl
from jax.experimental.pallas import tpu as pltpu
from jax.experimental.pallas import tpu_sc as plsc

N = 10000
E = 320000
D_IN = 128
D_H = 256
HALF = 128
L = 4

NC, NS = 2, 16
CHUNK = 128
E_PAD = 327680
NCHUNK = E_PAD // CHUNK
CPT = NCHUNK // NS
CPW = NCHUNK // (NC * NS)
ACC_ROWS = N + 112
ZROWS = ACC_ROWS // NS
SUP = 16

_mesh = plsc.VectorSubcoreMesh(core_axis_name="c", subcore_axis_name="s")


def _deg_body(dstc_hbm, ones_hbm, zeros_hbm, out_hbm, acc, dst_v, ones_v):
    c = lax.axis_index("c")
    s = lax.axis_index("s")
    pltpu.sync_copy(zeros_hbm.at[pl.ds(s * ZROWS, ZROWS)],
                    acc.at[pl.ds(s * ZROWS, ZROWS)])
    pltpu.sync_copy(ones_hbm, ones_v)
    w = c * NS + s
    base = w * CPW
    plsc.subcore_barrier()

    def body(g, carry):
        pltpu.sync_copy(dstc_hbm.at[pl.ds(base + g * SUP, SUP)], dst_v)
        for j in range(SUP):
            pltpu.sync_copy(ones_v, acc.at[dst_v.at[j]], add=True)
        return carry

    lax.fori_loop(0, CPW // SUP, body, 0)
    plsc.subcore_barrier()
    pltpu.sync_copy(acc.at[pl.ds(s * ZROWS, ZROWS)],
                    out_hbm.at[c, pl.ds(s * ZROWS, ZROWS)])


_deg = pl.kernel(
    _deg_body,
    out_type=jax.ShapeDtypeStruct((NC, ACC_ROWS, HALF), jnp.float32),
    name="gcn_deg_sc",
    mesh=_mesh,
    scratch_types=[
        pltpu.VMEM_SHARED((ACC_ROWS, HALF), jnp.float32),
        pltpu.VMEM((SUP, CHUNK), jnp.int32),
        pltpu.VMEM((CHUNK, HALF), jnp.float32),
    ],
)


def _agg_body(hwp_hbm, srcc_hbm, dstc_hbm, zeros_hbm, out_hbm,
              acc, src_v, dst_v, stage0, stage1, sem0, sem1, sem2, sem3):
    c = lax.axis_index("c")
    s = lax.axis_index("s")
    pltpu.sync_copy(zeros_hbm.at[pl.ds(s * ZROWS, ZROWS)],
                    acc.at[pl.ds(s * ZROWS, ZROWS)])
    base = s * CPT
    plsc.subcore_barrier()
    stages = (stage0, stage1)
    gsems = (sem0, sem1)
    ssems = (sem2, sem3)

    def body(g, carry):
        pltpu.sync_copy(srcc_hbm.at[c, pl.ds(base + g * SUP, SUP)], src_v)
        pltpu.sync_copy(dstc_hbm.at[pl.ds(base + g * SUP, SUP)], dst_v)
        pend = pltpu.async_copy(hwp_hbm.at[src_v.at[0]], stages[0], gsems[0])
        scats = [None, None]
        for j in range(SUP):
            k = j % 2
            if j + 1 < SUP:
                if scats[1 - k] is not None:
                    scats[1 - k].wait()
                nxt = pltpu.async_copy(hwp_hbm.at[src_v.at[j + 1]],
                                       stages[1 - k], gsems[1 - k])
            pend.wait()
            scats[k] = pltpu.async_copy(stages[k], acc.at[dst_v.at[j]],
                                        ssems[k], add=True)
            if j + 1 < SUP:
                pend = nxt
        scats[0].wait()
        scats[1].wait()
        return carry

    lax.fori_loop(0, CPT // SUP, body, 0)
    plsc.subcore_barrier()
    pltpu.sync_copy(acc.at[pl.ds(s * ZROWS, ZROWS)],
                    out_hbm.at[c, pl.ds(s * ZROWS, ZROWS)])


_agg = pl.kernel(
    _agg_body,
    out_type=jax.ShapeDtypeStruct((NC, ACC_ROWS, HALF), jnp.float32),
    name="gcn_agg_sc",
    mesh=_mesh,
    scratch_types=[
        pltpu.VMEM_SHARED((ACC_ROWS, HALF), jnp.float32),
        pltpu.VMEM((SUP, CHUNK), jnp.int32),
        pltpu.VMEM((SUP, CHUNK), jnp.int32),
        pltpu.VMEM((CHUNK, HALF), jnp.float32),
        pltpu.VMEM((CHUNK, HALF), jnp.float32),
        pltpu.SemaphoreType.DMA,
        pltpu.SemaphoreType.DMA,
        pltpu.SemaphoreType.DMA,
        pltpu.SemaphoreType.DMA,
    ],
)


_BLK = 1000


def _split_store(o_ref, hw):
    o_ref[0, :, :] = hw[:, :HALF]
    o_ref[1, :, :] = hw[:, HALF:]


def _proj_pre_body(x_ref, wp_ref, bp_ref, w0_ref, dsum_ref, h_ref, hwp_ref):
    h = jnp.maximum(
        jnp.dot(x_ref[...], wp_ref[...], preferred_element_type=jnp.float32)
        + bp_ref[...], 0.0)
    h_ref[...] = h
    dinv = lax.rsqrt(dsum_ref[...])
    hw = jnp.dot(h, w0_ref[...], preferred_element_type=jnp.float32) * dinv
    _split_store(hwp_ref, hw)


def _proj_pre(x, Wp, bp, W0, dsum):
    return pl.pallas_call(
        _proj_pre_body,
        grid=(N // _BLK,),
        in_specs=[
            pl.BlockSpec((_BLK, D_IN), lambda i: (i, 0)),
            pl.BlockSpec((D_IN, D_H), lambda i: (0, 0)),
            pl.BlockSpec((1, D_H), lambda i: (0, 0)),
            pl.BlockSpec((D_H, D_H), lambda i: (0, 0)),
            pl.BlockSpec((_BLK, 1), lambda i: (i, 0)),
        ],
        out_specs=(
            pl.BlockSpec((_BLK, D_H), lambda i: (i, 0)),
            pl.BlockSpec((2, _BLK, HALF), lambda i: (0, i, 0)),
        ),
        out_shape=(
            jax.ShapeDtypeStruct((N, D_H), jnp.float32),
            jax.ShapeDtypeStruct((2, N, HALF), jnp.float32),
        ),
    )(x, Wp, bp.reshape(1, -1), W0, dsum)


def _epilogue(agg_ref, hwp_ref, dsum_ref, bnsc_ref, bnsh_ref, lnw_ref,
              lnb_ref, hprev_ref):
    dinv = lax.rsqrt(dsum_ref[...])
    agg = jnp.concatenate([agg_ref[0], agg_ref[1]], axis=1)
    hwp = jnp.concatenate([hwp_ref[0], hwp_ref[1]], axis=1)
    t = dinv * (agg + hwp) * bnsc_ref[...] + bnsh_ref[...]
    mu = jnp.mean(t, axis=1, keepdims=True)
    var = jnp.mean(t * t, axis=1, keepdims=True) - mu * mu
    t = (t - mu) * lax.rsqrt(var + 1e-5) * lnw_ref[...] + lnb_ref[...]
    return jnp.maximum(t, 0.0) + hprev_ref[...], dinv


def _post_pre_body(agg_ref, hwp_ref, dsum_ref, bnsc_ref, bnsh_ref, lnw_ref,
                   lnb_ref, hprev_ref, wn_ref, h_ref, hwpn_ref):
    h, dinv = _epilogue(agg_ref, hwp_ref, dsum_ref, bnsc_ref, bnsh_ref,
                        lnw_ref, lnb_ref, hprev_ref)
    h_ref[...] = h
    hw = jnp.dot(h, wn_ref[...], preferred_element_type=jnp.float32) * dinv
    _split_store(hwpn_ref, hw)


_EPI_SPECS = [
    pl.BlockSpec((2, _BLK, HALF), lambda i: (0, i, 0)),
    pl.BlockSpec((2, _BLK, HALF), lambda i: (0, i, 0)),
    pl.BlockSpec((_BLK, 1), lambda i: (i, 0)),
    pl.BlockSpec((1, D_H), lambda i: (0, 0)),
    pl.BlockSpec((1, D_H), lambda i: (0, 0)),
    pl.BlockSpec((1, D_H), lambda i: (0, 0)),
    pl.BlockSpec((1, D_H), lambda i: (0, 0)),
    pl.BlockSpec((_BLK, D_H), lambda i: (i, 0)),
]


def _post_pre(agg_pl, hwp_pl, dsum, bnsc, bnsh, lnw, lnb, hprev, Wn):
    return pl.pallas_call(
        _post_pre_body,
        grid=(N // _BLK,),
        in_specs=_EPI_SPECS + [pl.BlockSpec((D_H, D_H), lambda i: (0, 0))],
        out_specs=(
            pl.BlockSpec((_BLK, D_H), lambda i: (i, 0)),
            pl.BlockSpec((2, _BLK, HALF), lambda i: (0, i, 0)),
        ),
        out_shape=(
            jax.ShapeDtypeStruct((N, D_H), jnp.float32),
            jax.ShapeDtypeStruct((2, N, HALF), jnp.float32),
        ),
    )(agg_pl, hwp_pl, dsum, bnsc.reshape(1, -1), bnsh.reshape(1, -1),
      lnw.reshape(1, -1), lnb.reshape(1, -1), hprev, Wn)


def _post_head_body(agg_ref, hwp_ref, dsum_ref, bnsc_ref, bnsh_ref, lnw_ref,
                    lnb_ref, hprev_ref, w1_ref, b1_ref, w2_ref, b2_ref,
                    wo_ref, bo_ref, o_ref):
    h, _ = _epilogue(agg_ref, hwp_ref, dsum_ref, bnsc_ref, bnsh_ref,
                     lnw_ref, lnb_ref, hprev_ref)
    t = jnp.maximum(
        jnp.dot(h, w1_ref[...], preferred_element_type=jnp.float32)
        + b1_ref[...], 0.0)
    t = jnp.dot(t, w2_ref[...], preferred_element_type=jnp.float32) + b2_ref[...]
    o = jnp.dot(t, wo_ref[...], preferred_element_type=jnp.float32) + bo_ref[...]
    m = jnp.max(o, axis=1, keepdims=True)
    lse = jnp.log(jnp.sum(jnp.exp(o - m), axis=1, keepdims=True))
    o_ref[...] = o - m - lse


def _post_head(agg_pl, hwp_pl, dsum, bnsc, bnsh, lnw, lnb, hprev,
               W1, b1, W2, b2, Wo, bo):
    d_out = Wo.shape[1]
    return pl.pallas_call(
        _post_head_body,
        grid=(N // _BLK,),
        in_specs=_EPI_SPECS + [
            pl.BlockSpec((D_H, 2 * D_H), lambda i: (0, 0)),
            pl.BlockSpec((1, 2 * D_H), lambda i: (0, 0)),
            pl.BlockSpec((2 * D_H, D_H), lambda i: (0, 0)),
            pl.BlockSpec((1, D_H), lambda i: (0, 0)),
            pl.BlockSpec((D_H, d_out), lambda i: (0, 0)),
            pl.BlockSpec((1, d_out), lambda i: (0, 0)),
        ],
        out_specs=pl.BlockSpec((_BLK, d_out), lambda i: (i, 0)),
        out_shape=jax.ShapeDtypeStruct((N, d_out), jnp.float32),
    )(agg_pl, hwp_pl, dsum, bnsc.reshape(1, -1), bnsh.reshape(1, -1),
      lnw.reshape(1, -1), lnb.reshape(1, -1), hprev,
      W1, b1.reshape(1, -1), W2, b2.reshape(1, -1), Wo, bo.reshape(1, -1))


def kernel(x, edge_index, Wp, bp, convW, convb, bn_w, bn_b, bn_mean, bn_var,
           ln_w, ln_b, mlpW1, mlpb1, mlpW2, mlpb2, outW, outb):
    src = edge_index[0]
    dst = edge_index[1]
    pad = E_PAD - E
    src_pad = jnp.concatenate(
        [src, (jnp.arange(pad, dtype=jnp.int32) * 97) % N])
    dst_pad = jnp.concatenate(
        [dst, N + (jnp.arange(pad, dtype=jnp.int32) % 16)])
    dst_chunks = dst_pad.reshape(NCHUNK, CHUNK)
    src_chunks2 = jnp.stack([src_pad, src_pad + N]).reshape(NC, NCHUNK, CHUNK)
    ones128 = jnp.ones((CHUNK, HALF), jnp.float32)
    zeros128 = jnp.zeros((ACC_ROWS, HALF), jnp.float32)

    deg_pl = _deg(dst_chunks, ones128, zeros128)
    dsum = deg_pl[0, :N, 0:1] + deg_pl[1, :N, 0:1] + 1.0

    bn_sc = bn_w / jnp.sqrt(bn_var + 1e-5)
    bn_sh = (convb - bn_mean) * bn_sc + bn_b

    h, hwp_pl = _proj_pre(x, Wp, bp, convW[0], dsum)
    for i in range(L):
        agg_pl = _agg(hwp_pl.reshape(2 * N, HALF), src_chunks2, dst_chunks,
                      zeros128)
        if i < L - 1:
            h, hwp_pl = _post_pre(agg_pl, hwp_pl, dsum, bn_sc[i], bn_sh[i],
                                  ln_w[i], ln_b[i], h, convW[i + 1])
        else:
            out = _post_head(agg_pl, hwp_pl, dsum, bn_sc[i], bn_sh[i],
                             ln_w[i], ln_b[i], h, mlpW1, mlpb1, mlpW2, mlpb2,
                             outW, outb)
    return out


# speedup vs baseline: 17.0323x; 1.0063x over previous
import functools

import jax
import jax.numpy as jnp
from jax import lax
from jax.experimental import pallas as pl
from jax.experimental.pallas import tpu as pltpu
from jax.experimental.pallas import tpu_sc as plsc

N = 10000
E = 320000
D_IN = 128
D_H = 256
HALF = 128
L = 4

NC, NS = 2, 16
CHUNK = 128
E_PAD = 327680
NCHUNK = E_PAD // CHUNK
CPT = NCHUNK // NS
CPW = NCHUNK // (NC * NS)
ACC_ROWS = N + 112
ZROWS = ACC_ROWS // NS
SUP = 16

_mesh = plsc.VectorSubcoreMesh(core_axis_name="c", subcore_axis_name="s")


def _deg_body(dstc_hbm, ones_hbm, zeros_hbm, out_hbm, acc, dst_v, ones_v):
    c = lax.axis_index("c")
    s = lax.axis_index("s")
    pltpu.sync_copy(zeros_hbm.at[pl.ds(s * ZROWS, ZROWS)],
                    acc.at[pl.ds(s * ZROWS, ZROWS)])
    pltpu.sync_copy(ones_hbm, ones_v)
    w = c * NS + s
    base = w * CPW
    plsc.subcore_barrier()

    def body(g, carry):
        pltpu.sync_copy(dstc_hbm.at[pl.ds(base + g * SUP, SUP)], dst_v)
        for j in range(SUP):
            pltpu.sync_copy(ones_v, acc.at[dst_v.at[j]], add=True)
        return carry

    lax.fori_loop(0, CPW // SUP, body, 0)
    plsc.subcore_barrier()
    pltpu.sync_copy(acc.at[pl.ds(s * ZROWS, ZROWS)],
                    out_hbm.at[c, pl.ds(s * ZROWS, ZROWS)])


_deg = pl.kernel(
    _deg_body,
    out_type=jax.ShapeDtypeStruct((NC, ACC_ROWS, HALF), jnp.float32),
    name="gcn_deg_sc",
    mesh=_mesh,
    scratch_types=[
        pltpu.VMEM_SHARED((ACC_ROWS, HALF), jnp.float32),
        pltpu.VMEM((SUP, CHUNK), jnp.int32),
        pltpu.VMEM((CHUNK, HALF), jnp.float32),
    ],
)


def _agg_body(hwp_hbm, srcc_hbm, dstc_hbm, out_hbm,
              acc, src_v, dst_v, stage0, stage1, sem0, sem1, sem2, sem3):
    c = lax.axis_index("c")
    s = lax.axis_index("s")
    pltpu.sync_copy(hwp_hbm.at[pl.ds(c * ACC_ROWS + s * ZROWS, ZROWS)],
                    acc.at[pl.ds(s * ZROWS, ZROWS)])
    base = s * CPT
    plsc.subcore_barrier()
    stages = (stage0, stage1)
    gsems = (sem0, sem1)
    ssems = (sem2, sem3)

    def body(g, carry):
        pltpu.sync_copy(srcc_hbm.at[c, pl.ds(base + g * SUP, SUP)], src_v)
        pltpu.sync_copy(dstc_hbm.at[pl.ds(base + g * SUP, SUP)], dst_v)
        pend = pltpu.async_copy(hwp_hbm.at[src_v.at[0]], stages[0], gsems[0])
        scats = [None, None]
        for j in range(SUP):
            k = j % 2
            if j + 1 < SUP:
                if scats[1 - k] is not None:
                    scats[1 - k].wait()
                nxt = pltpu.async_copy(hwp_hbm.at[src_v.at[j + 1]],
                                       stages[1 - k], gsems[1 - k])
            pend.wait()
            scats[k] = pltpu.async_copy(stages[k], acc.at[dst_v.at[j]],
                                        ssems[k], add=True)
            if j + 1 < SUP:
                pend = nxt
        scats[0].wait()
        scats[1].wait()
        return carry

    lax.fori_loop(0, CPT // SUP, body, 0)
    plsc.subcore_barrier()
    pltpu.sync_copy(acc.at[pl.ds(s * ZROWS, ZROWS)],
                    out_hbm.at[c, pl.ds(s * ZROWS, ZROWS)])


_agg = pl.kernel(
    _agg_body,
    out_type=jax.ShapeDtypeStruct((NC, ACC_ROWS, HALF), jnp.float32),
    name="gcn_agg_sc",
    mesh=_mesh,
    scratch_types=[
        pltpu.VMEM_SHARED((ACC_ROWS, HALF), jnp.float32),
        pltpu.VMEM((SUP, CHUNK), jnp.int32),
        pltpu.VMEM((SUP, CHUNK), jnp.int32),
        pltpu.VMEM((CHUNK, HALF), jnp.float32),
        pltpu.VMEM((CHUNK, HALF), jnp.float32),
        pltpu.SemaphoreType.DMA,
        pltpu.SemaphoreType.DMA,
        pltpu.SemaphoreType.DMA,
        pltpu.SemaphoreType.DMA,
    ],
)


_BLK = 1000


def _split_store(o_ref, hw):
    o_ref[0, :, :] = hw[:, :HALF]
    o_ref[1, :, :] = hw[:, HALF:]


def _proj_pre_body(x_ref, wp_ref, bp_ref, w0_ref, dsum_ref, h_ref, hwp_ref):
    h = jnp.maximum(
        jnp.dot(x_ref[...], wp_ref[...], preferred_element_type=jnp.float32)
        + bp_ref[...], 0.0)
    h_ref[...] = h
    dinv = lax.rsqrt(dsum_ref[...])
    hw = jnp.dot(h, w0_ref[...], preferred_element_type=jnp.float32) * dinv
    _split_store(hwp_ref, hw)


def _proj_pre(x, Wp, bp, W0, dsum):
    return pl.pallas_call(
        _proj_pre_body,
        grid=(N // _BLK,),
        in_specs=[
            pl.BlockSpec((_BLK, D_IN), lambda i: (i, 0)),
            pl.BlockSpec((D_IN, D_H), lambda i: (0, 0)),
            pl.BlockSpec((1, D_H), lambda i: (0, 0)),
            pl.BlockSpec((D_H, D_H), lambda i: (0, 0)),
            pl.BlockSpec((_BLK, 1), lambda i: (i, 0)),
        ],
        out_specs=(
            pl.BlockSpec((_BLK, D_H), lambda i: (i, 0)),
            pl.BlockSpec((2, _BLK, HALF), lambda i: (0, i, 0)),
        ),
        out_shape=(
            jax.ShapeDtypeStruct((N, D_H), jnp.float32),
            jax.ShapeDtypeStruct((2, ACC_ROWS, HALF), jnp.float32),
        ),
    )(x, Wp, bp.reshape(1, -1), W0, dsum)


def _epilogue(agg_ref, dsum_ref, bnsc_ref, bnsh_ref, lnw_ref,
              lnb_ref, hprev_ref):
    dinv = lax.rsqrt(dsum_ref[...])
    agg = jnp.concatenate([agg_ref[0], agg_ref[1]], axis=1)
    t = dinv * agg * bnsc_ref[...] + bnsh_ref[...]
    mu = jnp.mean(t, axis=1, keepdims=True)
    var = jnp.mean(t * t, axis=1, keepdims=True) - mu * mu
    t = (t - mu) * lax.rsqrt(var + 1e-5) * lnw_ref[...] + lnb_ref[...]
    return jnp.maximum(t, 0.0) + hprev_ref[...], dinv


def _post_pre_body(agg_ref, dsum_ref, bnsc_ref, bnsh_ref, lnw_ref,
                   lnb_ref, hprev_ref, wn_ref, h_ref, hwpn_ref):
    h, dinv = _epilogue(agg_ref, dsum_ref, bnsc_ref, bnsh_ref,
                        lnw_ref, lnb_ref, hprev_ref)
    h_ref[...] = h
    hw = jnp.dot(h, wn_ref[...], preferred_element_type=jnp.float32) * dinv
    _split_store(hwpn_ref, hw)


_EPI_SPECS = [
    pl.BlockSpec((2, _BLK, HALF), lambda i: (0, i, 0)),
    pl.BlockSpec((_BLK, 1), lambda i: (i, 0)),
    pl.BlockSpec((1, D_H), lambda i: (0, 0)),
    pl.BlockSpec((1, D_H), lambda i: (0, 0)),
    pl.BlockSpec((1, D_H), lambda i: (0, 0)),
    pl.BlockSpec((1, D_H), lambda i: (0, 0)),
    pl.BlockSpec((_BLK, D_H), lambda i: (i, 0)),
]


def _post_pre(agg_pl, dsum, bnsc, bnsh, lnw, lnb, hprev, Wn):
    return pl.pallas_call(
        _post_pre_body,
        grid=(N // _BLK,),
        in_specs=_EPI_SPECS + [pl.BlockSpec((D_H, D_H), lambda i: (0, 0))],
        out_specs=(
            pl.BlockSpec((_BLK, D_H), lambda i: (i, 0)),
            pl.BlockSpec((2, _BLK, HALF), lambda i: (0, i, 0)),
        ),
        out_shape=(
            jax.ShapeDtypeStruct((N, D_H), jnp.float32),
            jax.ShapeDtypeStruct((2, ACC_ROWS, HALF), jnp.float32),
        ),
    )(agg_pl, dsum, bnsc.reshape(1, -1), bnsh.reshape(1, -1),
      lnw.reshape(1, -1), lnb.reshape(1, -1), hprev, Wn)


def _post_head_body(agg_ref, dsum_ref, bnsc_ref, bnsh_ref, lnw_ref,
                    lnb_ref, hprev_ref, w1_ref, b1_ref, w2_ref, b2_ref,
                    wo_ref, bo_ref, o_ref):
    h, _ = _epilogue(agg_ref, dsum_ref, bnsc_ref, bnsh_ref,
                     lnw_ref, lnb_ref, hprev_ref)
    t = jnp.maximum(
        jnp.dot(h, w1_ref[...], preferred_element_type=jnp.float32)
        + b1_ref[...], 0.0)
    t = jnp.dot(t, w2_ref[...], preferred_element_type=jnp.float32) + b2_ref[...]
    o = jnp.dot(t, wo_ref[...], preferred_element_type=jnp.float32) + bo_ref[...]
    m = jnp.max(o, axis=1, keepdims=True)
    lse = jnp.log(jnp.sum(jnp.exp(o - m), axis=1, keepdims=True))
    o_ref[...] = o - m - lse


def _post_head(agg_pl, dsum, bnsc, bnsh, lnw, lnb, hprev,
               W1, b1, W2, b2, Wo, bo):
    d_out = Wo.shape[1]
    return pl.pallas_call(
        _post_head_body,
        grid=(N // _BLK,),
        in_specs=_EPI_SPECS + [
            pl.BlockSpec((D_H, 2 * D_H), lambda i: (0, 0)),
            pl.BlockSpec((1, 2 * D_H), lambda i: (0, 0)),
            pl.BlockSpec((2 * D_H, D_H), lambda i: (0, 0)),
            pl.BlockSpec((1, D_H), lambda i: (0, 0)),
            pl.BlockSpec((D_H, d_out), lambda i: (0, 0)),
            pl.BlockSpec((1, d_out), lambda i: (0, 0)),
        ],
        out_specs=pl.BlockSpec((_BLK, d_out), lambda i: (i, 0)),
        out_shape=jax.ShapeDtypeStruct((N, d_out), jnp.float32),
    )(agg_pl, dsum, bnsc.reshape(1, -1), bnsh.reshape(1, -1),
      lnw.reshape(1, -1), lnb.reshape(1, -1), hprev,
      W1, b1.reshape(1, -1), W2, b2.reshape(1, -1), Wo, bo.reshape(1, -1))


def kernel(x, edge_index, Wp, bp, convW, convb, bn_w, bn_b, bn_mean, bn_var,
           ln_w, ln_b, mlpW1, mlpb1, mlpW2, mlpb2, outW, outb):
    src = edge_index[0]
    dst = edge_index[1]
    pad = E_PAD - E
    src_pad = jnp.concatenate(
        [src, (jnp.arange(pad, dtype=jnp.int32) * 97) % N])
    dst_pad = jnp.concatenate(
        [dst, N + (jnp.arange(pad, dtype=jnp.int32) % 16)])
    dst_chunks = dst_pad.reshape(NCHUNK, CHUNK)
    src_chunks2 = jnp.stack([src_pad, src_pad + ACC_ROWS]).reshape(
        NC, NCHUNK, CHUNK)
    ones128 = jnp.ones((CHUNK, HALF), jnp.float32)
    zeros128 = jnp.zeros((ACC_ROWS, HALF), jnp.float32)

    deg_pl = _deg(dst_chunks, ones128, zeros128)
    dsum = deg_pl[0, :N, 0:1] + deg_pl[1, :N, 0:1] + 1.0

    bn_sc = bn_w / jnp.sqrt(bn_var + 1e-5)
    bn_sh = (convb - bn_mean) * bn_sc + bn_b

    h, hwp_pl = _proj_pre(x, Wp, bp, convW[0], dsum)
    for i in range(L):
        agg_pl = _agg(hwp_pl.reshape(2 * ACC_ROWS, HALF), src_chunks2,
                      dst_chunks)
        if i < L - 1:
            h, hwp_pl = _post_pre(agg_pl, dsum, bn_sc[i], bn_sh[i],
                                  ln_w[i], ln_b[i], h, convW[i + 1])
        else:
            out = _post_head(agg_pl, dsum, bn_sc[i], bn_sh[i],
                             ln_w[i], ln_b[i], h, mlpW1, mlpb1, mlpW2, mlpb2,
                             outW, outb)
    return out


# speedup vs baseline: 18.0112x; 1.0575x over previous
import functools

import jax
import jax.numpy as jnp
from jax import lax
from jax.experimental import pallas as pl
from jax.experimental.pallas import tpu as pltpu
from jax.experimental.pallas import tpu_sc as plsc

N = 10000
E = 320000
D_IN = 128
D_H = 256
HALF = 128
L = 4

NC, NS = 2, 16
CHUNK = 128
E_PAD = 327680
NCHUNK = E_PAD // CHUNK
CPT = NCHUNK // NS
CPW = NCHUNK // (NC * NS)
ACC_ROWS = N + 112
ZROWS = ACC_ROWS // NS
SUP = 32

_mesh = plsc.VectorSubcoreMesh(core_axis_name="c", subcore_axis_name="s")


def _deg_body(dstc_hbm, ones_hbm, zeros_hbm, out_hbm, acc, dst_v, ones_v):
    c = lax.axis_index("c")
    s = lax.axis_index("s")
    pltpu.sync_copy(zeros_hbm.at[pl.ds(s * ZROWS, ZROWS)],
                    acc.at[pl.ds(s * ZROWS, ZROWS)])
    pltpu.sync_copy(ones_hbm, ones_v)
    w = c * NS + s
    base = w * CPW
    plsc.subcore_barrier()

    def body(g, carry):
        pltpu.sync_copy(dstc_hbm.at[pl.ds(base + g * SUP, SUP)], dst_v)
        for j in range(SUP):
            pltpu.sync_copy(ones_v, acc.at[dst_v.at[j]], add=True)
        return carry

    lax.fori_loop(0, CPW // SUP, body, 0)
    plsc.subcore_barrier()
    pltpu.sync_copy(acc.at[pl.ds(s * ZROWS, ZROWS)],
                    out_hbm.at[c, pl.ds(s * ZROWS, ZROWS)])


_deg = pl.kernel(
    _deg_body,
    out_type=jax.ShapeDtypeStruct((NC, ACC_ROWS, HALF), jnp.float32),
    name="gcn_deg_sc",
    mesh=_mesh,
    scratch_types=[
        pltpu.VMEM_SHARED((ACC_ROWS, HALF), jnp.float32),
        pltpu.VMEM((SUP, CHUNK), jnp.int32),
        pltpu.VMEM((CHUNK, HALF), jnp.float32),
    ],
)


def _agg_body(hwp_hbm, srcc_hbm, dstc_hbm, out_hbm,
              acc, src_v, dst_v, stage0, stage1, sem0, sem1, sem2, sem3):
    c = lax.axis_index("c")
    s = lax.axis_index("s")
    pltpu.sync_copy(hwp_hbm.at[pl.ds(c * ACC_ROWS + s * ZROWS, ZROWS)],
                    acc.at[pl.ds(s * ZROWS, ZROWS)])
    base = s * CPT
    plsc.subcore_barrier()
    stages = (stage0, stage1)
    gsems = (sem0, sem1)
    ssems = (sem2, sem3)

    def body(g, carry):
        pltpu.sync_copy(srcc_hbm.at[c, pl.ds(base + g * SUP, SUP)], src_v)
        pltpu.sync_copy(dstc_hbm.at[pl.ds(base + g * SUP, SUP)], dst_v)
        pend = pltpu.async_copy(hwp_hbm.at[src_v.at[0]], stages[0], gsems[0])
        scats = [None, None]
        for j in range(SUP):
            k = j % 2
            if j + 1 < SUP:
                if scats[1 - k] is not None:
                    scats[1 - k].wait()
                nxt = pltpu.async_copy(hwp_hbm.at[src_v.at[j + 1]],
                                       stages[1 - k], gsems[1 - k])
            pend.wait()
            scats[k] = pltpu.async_copy(stages[k], acc.at[dst_v.at[j]],
                                        ssems[k], add=True)
            if j + 1 < SUP:
                pend = nxt
        scats[0].wait()
        scats[1].wait()
        return carry

    lax.fori_loop(0, CPT // SUP, body, 0)
    plsc.subcore_barrier()
    pltpu.sync_copy(acc.at[pl.ds(s * ZROWS, ZROWS)],
                    out_hbm.at[c, pl.ds(s * ZROWS, ZROWS)])


_agg = pl.kernel(
    _agg_body,
    out_type=jax.ShapeDtypeStruct((NC, ACC_ROWS, HALF), jnp.float32),
    name="gcn_agg_sc",
    mesh=_mesh,
    scratch_types=[
        pltpu.VMEM_SHARED((ACC_ROWS, HALF), jnp.float32),
        pltpu.VMEM((SUP, CHUNK), jnp.int32),
        pltpu.VMEM((SUP, CHUNK), jnp.int32),
        pltpu.VMEM((CHUNK, HALF), jnp.float32),
        pltpu.VMEM((CHUNK, HALF), jnp.float32),
        pltpu.SemaphoreType.DMA,
        pltpu.SemaphoreType.DMA,
        pltpu.SemaphoreType.DMA,
        pltpu.SemaphoreType.DMA,
    ],
)


_BLK = 1000


def _split_store(o_ref, hw):
    o_ref[0, :, :] = hw[:, :HALF]
    o_ref[1, :, :] = hw[:, HALF:]


def _proj_pre_body(x_ref, wp_ref, bp_ref, w0_ref, dsum_ref, h_ref, hwp_ref):
    h = jnp.maximum(
        jnp.dot(x_ref[...], wp_ref[...], preferred_element_type=jnp.float32)
        + bp_ref[...], 0.0)
    h_ref[...] = h
    dinv = lax.rsqrt(dsum_ref[...])
    hw = jnp.dot(h, w0_ref[...], preferred_element_type=jnp.float32) * dinv
    _split_store(hwp_ref, hw)


def _proj_pre(x, Wp, bp, W0, dsum):
    return pl.pallas_call(
        _proj_pre_body,
        grid=(N // _BLK,),
        in_specs=[
            pl.BlockSpec((_BLK, D_IN), lambda i: (i, 0)),
            pl.BlockSpec((D_IN, D_H), lambda i: (0, 0)),
            pl.BlockSpec((1, D_H), lambda i: (0, 0)),
            pl.BlockSpec((D_H, D_H), lambda i: (0, 0)),
            pl.BlockSpec((_BLK, 1), lambda i: (i, 0)),
        ],
        out_specs=(
            pl.BlockSpec((_BLK, D_H), lambda i: (i, 0)),
            pl.BlockSpec((2, _BLK, HALF), lambda i: (0, i, 0)),
        ),
        out_shape=(
            jax.ShapeDtypeStruct((N, D_H), jnp.float32),
            jax.ShapeDtypeStruct((2, ACC_ROWS, HALF), jnp.float32),
        ),
    )(x, Wp, bp.reshape(1, -1), W0, dsum)


def _epilogue(agg_ref, dsum_ref, bnsc_ref, bnsh_ref, lnw_ref,
              lnb_ref, hprev_ref):
    dinv = lax.rsqrt(dsum_ref[...])
    agg = jnp.concatenate([agg_ref[0], agg_ref[1]], axis=1)
    t = dinv * agg * bnsc_ref[...] + bnsh_ref[...]
    mu = jnp.mean(t, axis=1, keepdims=True)
    var = jnp.mean(t * t, axis=1, keepdims=True) - mu * mu
    t = (t - mu) * lax.rsqrt(var + 1e-5) * lnw_ref[...] + lnb_ref[...]
    return jnp.maximum(t, 0.0) + hprev_ref[...], dinv


def _post_pre_body(agg_ref, dsum_ref, bnsc_ref, bnsh_ref, lnw_ref,
                   lnb_ref, hprev_ref, wn_ref, h_ref, hwpn_ref):
    h, dinv = _epilogue(agg_ref, dsum_ref, bnsc_ref, bnsh_ref,
                        lnw_ref, lnb_ref, hprev_ref)
    h_ref[...] = h
    hw = jnp.dot(h, wn_ref[...], preferred_element_type=jnp.float32) * dinv
    _split_store(hwpn_ref, hw)


_EPI_SPECS = [
    pl.BlockSpec((2, _BLK, HALF), lambda i: (0, i, 0)),
    pl.BlockSpec((_BLK, 1), lambda i: (i, 0)),
    pl.BlockSpec((1, D_H), lambda i: (0, 0)),
    pl.BlockSpec((1, D_H), lambda i: (0, 0)),
    pl.BlockSpec((1, D_H), lambda i: (0, 0)),
    pl.BlockSpec((1, D_H), lambda i: (0, 0)),
    pl.BlockSpec((_BLK, D_H), lambda i: (i, 0)),
]


def _post_pre(agg_pl, dsum, bnsc, bnsh, lnw, lnb, hprev, Wn):
    return pl.pallas_call(
        _post_pre_body,
        grid=(N // _BLK,),
        in_specs=_EPI_SPECS + [pl.BlockSpec((D_H, D_H), lambda i: (0, 0))],
        out_specs=(
            pl.BlockSpec((_BLK, D_H), lambda i: (i, 0)),
            pl.BlockSpec((2, _BLK, HALF), lambda i: (0, i, 0)),
        ),
        out_shape=(
            jax.ShapeDtypeStruct((N, D_H), jnp.float32),
            jax.ShapeDtypeStruct((2, ACC_ROWS, HALF), jnp.float32),
        ),
    )(agg_pl, dsum, bnsc.reshape(1, -1), bnsh.reshape(1, -1),
      lnw.reshape(1, -1), lnb.reshape(1, -1), hprev, Wn)


def _post_head_body(agg_ref, dsum_ref, bnsc_ref, bnsh_ref, lnw_ref,
                    lnb_ref, hprev_ref, w1_ref, b1_ref, w2_ref, b2_ref,
                    wo_ref, bo_ref, o_ref):
    h, _ = _epilogue(agg_ref, dsum_ref, bnsc_ref, bnsh_ref,
                     lnw_ref, lnb_ref, hprev_ref)
    t = jnp.maximum(
        jnp.dot(h, w1_ref[...], preferred_element_type=jnp.float32)
        + b1_ref[...], 0.0)
    t = jnp.dot(t, w2_ref[...], preferred_element_type=jnp.float32) + b2_ref[...]
    o = jnp.dot(t, wo_ref[...], preferred_element_type=jnp.float32) + bo_ref[...]
    m = jnp.max(o, axis=1, keepdims=True)
    lse = jnp.log(jnp.sum(jnp.exp(o - m), axis=1, keepdims=True))
    o_ref[...] = o - m - lse


def _post_head(agg_pl, dsum, bnsc, bnsh, lnw, lnb, hprev,
               W1, b1, W2, b2, Wo, bo):
    d_out = Wo.shape[1]
    return pl.pallas_call(
        _post_head_body,
        grid=(N // _BLK,),
        in_specs=_EPI_SPECS + [
            pl.BlockSpec((D_H, 2 * D_H), lambda i: (0, 0)),
            pl.BlockSpec((1, 2 * D_H), lambda i: (0, 0)),
            pl.BlockSpec((2 * D_H, D_H), lambda i: (0, 0)),
            pl.BlockSpec((1, D_H), lambda i: (0, 0)),
            pl.BlockSpec((D_H, d_out), lambda i: (0, 0)),
            pl.BlockSpec((1, d_out), lambda i: (0, 0)),
        ],
        out_specs=pl.BlockSpec((_BLK, d_out), lambda i: (i, 0)),
        out_shape=jax.ShapeDtypeStruct((N, d_out), jnp.float32),
    )(agg_pl, dsum, bnsc.reshape(1, -1), bnsh.reshape(1, -1),
      lnw.reshape(1, -1), lnb.reshape(1, -1), hprev,
      W1, b1.reshape(1, -1), W2, b2.reshape(1, -1), Wo, bo.reshape(1, -1))


def kernel(x, edge_index, Wp, bp, convW, convb, bn_w, bn_b, bn_mean, bn_var,
           ln_w, ln_b, mlpW1, mlpb1, mlpW2, mlpb2, outW, outb):
    src = edge_index[0]
    dst = edge_index[1]
    pad = E_PAD - E
    src_pad = jnp.concatenate(
        [src, (jnp.arange(pad, dtype=jnp.int32) * 97) % N])
    dst_pad = jnp.concatenate(
        [dst, N + (jnp.arange(pad, dtype=jnp.int32) % 16)])
    dst_chunks = dst_pad.reshape(NCHUNK, CHUNK)
    src_chunks2 = jnp.stack([src_pad, src_pad + ACC_ROWS]).reshape(
        NC, NCHUNK, CHUNK)
    ones128 = jnp.ones((CHUNK, HALF), jnp.float32)
    zeros128 = jnp.zeros((ACC_ROWS, HALF), jnp.float32)

    deg_pl = _deg(dst_chunks, ones128, zeros128)
    dsum = deg_pl[0, :N, 0:1] + deg_pl[1, :N, 0:1] + 1.0

    bn_sc = bn_w / jnp.sqrt(bn_var + 1e-5)
    bn_sh = (convb - bn_mean) * bn_sc + bn_b

    h, hwp_pl = _proj_pre(x, Wp, bp, convW[0], dsum)
    for i in range(L):
        agg_pl = _agg(hwp_pl.reshape(2 * ACC_ROWS, HALF), src_chunks2,
                      dst_chunks)
        if i < L - 1:
            h, hwp_pl = _post_pre(agg_pl, dsum, bn_sc[i], bn_sh[i],
                                  ln_w[i], ln_b[i], h, convW[i + 1])
        else:
            out = _post_head(agg_pl, dsum, bn_sc[i], bn_sh[i],
                             ln_w[i], ln_b[i], h, mlpW1, mlpb1, mlpW2, mlpb2,
                             outW, outb)
    return out
